# baseline (speedup 1.0000x reference)
import jax
import jax.numpy as jnp
from jax.experimental import pallas as pl
from jax.experimental.pallas import tpu as pltpu

_BM = 400


def _rowsum_support_kernel(adj_ref, x_ref, w_ref, r_ref, t_ref):
    rs = jnp.sum(adj_ref[...], axis=1, keepdims=True) + 1.0
    rinv = jnp.power(rs, -0.5)
    rinv = jnp.where(jnp.isinf(rinv), 0.0, rinv)
    support = jnp.dot(x_ref[...], w_ref[...], preferred_element_type=jnp.float32)
    r_ref[...] = rinv
    t_ref[...] = rinv * support


def _spmm_kernel(adj_ref, t_ref, t_blk_ref, r_ref, b_ref, out_ref):
    acc = jnp.dot(adj_ref[...], t_ref[...], preferred_element_type=jnp.float32)
    out_ref[...] = r_ref[...] * (acc + t_blk_ref[...]) + b_ref[...]


def kernel(input, adj, W, b):
    n, f_in = input.shape
    f_out = W.shape[1]
    grid = (n // _BM,)

    r, t = pl.pallas_call(
        _rowsum_support_kernel,
        grid=grid,
        in_specs=[
            pl.BlockSpec((_BM, n), lambda m: (m, 0)),
            pl.BlockSpec((_BM, f_in), lambda m: (m, 0)),
            pl.BlockSpec((f_in, f_out), lambda m: (0, 0)),
        ],
        out_specs=[
            pl.BlockSpec((_BM, 1), lambda m: (m, 0)),
            pl.BlockSpec((_BM, f_out), lambda m: (m, 0)),
        ],
        out_shape=[
            jax.ShapeDtypeStruct((n, 1), jnp.float32),
            jax.ShapeDtypeStruct((n, f_out), jnp.float32),
        ],
        compiler_params=pltpu.CompilerParams(
            dimension_semantics=("arbitrary",),
        ),
    )(adj, input, W)

    b2 = b.reshape(1, f_out)
    out = pl.pallas_call(
        _spmm_kernel,
        grid=grid,
        in_specs=[
            pl.BlockSpec((_BM, n), lambda m: (m, 0)),
            pl.BlockSpec((n, f_out), lambda m: (0, 0)),
            pl.BlockSpec((_BM, f_out), lambda m: (m, 0)),
            pl.BlockSpec((_BM, 1), lambda m: (m, 0)),
            pl.BlockSpec((1, f_out), lambda m: (0, 0)),
        ],
        out_specs=pl.BlockSpec((_BM, f_out), lambda m: (m, 0)),
        out_shape=jax.ShapeDtypeStruct((n, f_out), jnp.float32),
        compiler_params=pltpu.CompilerParams(
            dimension_semantics=("arbitrary",),
        ),
    )(adj, t, t, r, b2)
    return out


# speedup vs baseline: 1.0018x; 1.0018x over previous
import jax
import jax.numpy as jnp
from jax.experimental import pallas as pl
from jax.experimental.pallas import tpu as pltpu

_BM = 400


def _rowsum_support_kernel(adj_ref, x_ref, w_ref, r_ref, t_ref):
    rs = jnp.sum(adj_ref[...], axis=1, keepdims=True) + 1.0
    rinv = jnp.power(rs, -0.5)
    rinv = jnp.where(jnp.isinf(rinv), 0.0, rinv)
    support = jnp.dot(x_ref[...], w_ref[...], preferred_element_type=jnp.float32)
    r_ref[...] = rinv
    t_ref[...] = rinv * support


def _spmm_kernel(adj_ref, t_ref, t_blk_ref, r_ref, b_ref, out_ref):
    a16 = adj_ref[...].astype(jnp.bfloat16)
    t16 = t_ref[...].astype(jnp.bfloat16)
    acc = jnp.dot(a16, t16, preferred_element_type=jnp.float32)
    out_ref[...] = r_ref[...] * (acc + t_blk_ref[...]) + b_ref[...]


def kernel(input, adj, W, b):
    n, f_in = input.shape
    f_out = W.shape[1]
    grid = (n // _BM,)

    r, t = pl.pallas_call(
        _rowsum_support_kernel,
        grid=grid,
        in_specs=[
            pl.BlockSpec((_BM, n), lambda m: (m, 0)),
            pl.BlockSpec((_BM, f_in), lambda m: (m, 0)),
            pl.BlockSpec((f_in, f_out), lambda m: (0, 0)),
        ],
        out_specs=[
            pl.BlockSpec((_BM, 1), lambda m: (m, 0)),
            pl.BlockSpec((_BM, f_out), lambda m: (m, 0)),
        ],
        out_shape=[
            jax.ShapeDtypeStruct((n, 1), jnp.float32),
            jax.ShapeDtypeStruct((n, f_out), jnp.float32),
        ],
        compiler_params=pltpu.CompilerParams(
            dimension_semantics=("arbitrary",),
        ),
    )(adj, input, W)

    b2 = b.reshape(1, f_out)
    out = pl.pallas_call(
        _spmm_kernel,
        grid=grid,
        in_specs=[
            pl.BlockSpec((_BM, n), lambda m: (m, 0)),
            pl.BlockSpec((n, f_out), lambda m: (0, 0)),
            pl.BlockSpec((_BM, f_out), lambda m: (m, 0)),
            pl.BlockSpec((_BM, 1), lambda m: (m, 0)),
            pl.BlockSpec((1, f_out), lambda m: (0, 0)),
        ],
        out_specs=pl.BlockSpec((_BM, f_out), lambda m: (m, 0)),
        out_shape=jax.ShapeDtypeStruct((n, f_out), jnp.float32),
        compiler_params=pltpu.CompilerParams(
            dimension_semantics=("arbitrary",),
        ),
    )(adj, t, t, r, b2)
    return out
